# baseline (speedup 1.0000x reference)
import functools

import jax
import jax.numpy as jnp
import numpy as np
from jax import lax
from jax.experimental import pallas as pl
from jax.experimental.pallas import tpu as pltpu
from jax.experimental.pallas import tpu_sc as plsc

_EPS = 1e-05
_NUM_GRAPHS = 16
_N, _D, _E = 10000, 128, 320000
_NC, _NS, _L = 2, 16, 16
_NW = _NC * _NS
_EW = _E // _NW
_CHUNK = 40
_NCHUNK = _EW // _CHUNK
_NBUF = 3
_UNROLL = 2

_ROW_SCRATCH = [pltpu.VMEM((_CHUNK, _D), jnp.float32)] * (3 * _NBUF)


def _sc_mesh():
    return plsc.VectorSubcoreMesh(core_axis_name="c", subcore_axis_name="s",
                                  num_cores=_NC, num_subcores=_NS)


@functools.partial(
    pl.kernel,
    out_type=[jax.ShapeDtypeStruct((_E,), jnp.float32),
              jax.ShapeDtypeStruct((_E,), jnp.float32)],
    mesh=_sc_mesh(),
    scratch_types=[
        pltpu.VMEM((_NUM_GRAPHS,), jnp.int32),
        pltpu.VMEM((_NUM_GRAPHS,), jnp.int32),
        pltpu.VMEM((_EW,), jnp.int32),
        pltpu.VMEM((_EW,), jnp.int32),
        pltpu.VMEM((_EW,), jnp.int32),
        pltpu.VMEM((_EW,), jnp.float32),
        pltpu.VMEM((_EW,), jnp.float32),
        pltpu.VMEM((_EW,), jnp.float32),
        pltpu.VMEM((2 * _EW,), jnp.int32),
    ] + _ROW_SCRATCH + [
        pltpu.SemaphoreType.DMA,
    ] + [pltpu.SemaphoreType.DMA] * _NBUF,
    compiler_params=pltpu.CompilerParams(needs_layout_passes=False),
)
def _sc_dots(z_hbm, srcp_hbm, dstp_hbm, u_hbm, cnt_hbm, start_hbm,
             pos_hbm, negv_hbm,
             cnt_v, start_v, srci_v, dsti_v, negi_v, u_v, posacc, negacc,
             raw_v, *rest):
    rows = rest[:3 * _NBUF]
    sem_in = rest[3 * _NBUF]
    sems = rest[3 * _NBUF + 1:]
    bufs = tuple((rows[3 * b], rows[3 * b + 1], rows[3 * b + 2], sems[b])
                 for b in range(_NBUF))

    wid = lax.axis_index("s") * jnp.int32(_NC) + lax.axis_index("c")
    base = wid * jnp.int32(_EW)
    lane = lax.iota(jnp.int32, _L)
    lane0 = lane == 0
    lane1 = lane == 1
    even = (lane & jnp.int32(1)) == 0
    perms = {sh: lane ^ jnp.int32(sh) for sh in (1, 2, 4, 8)}

    pltpu.sync_copy(cnt_hbm, cnt_v)
    pltpu.sync_copy(start_hbm, start_v)
    col0 = jnp.zeros((_L,), jnp.int32)

    base2 = base * jnp.int32(2)
    cp_c = pltpu.async_copy(u_hbm.at[pl.ds(base, _EW)], u_v, sem_in)
    cp_s = pltpu.async_copy(srcp_hbm.at[pl.ds(base2, 2 * _EW)], raw_v, sem_in)
    cp_s.wait()

    lane2 = lane * jnp.int32(2)

    def ext_src(t, c):
        sl = pl.ds(t * jnp.int32(_L), _L)
        ridx = lane2 + jnp.full((_L,), t * jnp.int32(2 * _L), jnp.int32)
        srci_v[sl] = plsc.load_gather(raw_v, [ridx])
        return c

    lax.fori_loop(jnp.int32(0), jnp.int32(_EW // _L), ext_src, jnp.int32(0))
    cp_d = pltpu.async_copy(dstp_hbm.at[pl.ds(base2, 2 * _EW)], raw_v, sem_in)
    cp_d.wait()

    def ext_dst(t, c):
        sl = pl.ds(t * jnp.int32(_L), _L)
        ridx = lane2 + jnp.full((_L,), t * jnp.int32(2 * _L), jnp.int32)
        dsti_v[sl] = plsc.load_gather(raw_v, [ridx])
        return c

    lax.fori_loop(jnp.int32(0), jnp.int32(_EW // _L), ext_dst, jnp.int32(0))
    cp_c.wait()

    def neg_body(t, c):
        sl = pl.ds(t * jnp.int32(_L), _L)
        sv = srci_v[sl]
        g = jnp.zeros((_L,), jnp.int32)
        for bit in (8, 4, 2, 1):
            probe = g | jnp.int32(bit)
            vals = plsc.load_gather(start_v, [probe])
            g = jnp.where(vals <= sv, probe, g)
        cnt = plsc.load_gather(cnt_v, [g])
        st = plsc.load_gather(start_v, [g])
        r = (u_v[sl] * cnt.astype(jnp.float32)).astype(jnp.int32)
        negi_v[sl] = st + jnp.minimum(r, cnt - 1)
        return c

    lax.fori_loop(jnp.int32(0), jnp.int32(_EW // _L), neg_body, jnp.int32(0))

    def gather_descs(j, b):
        sr, dr, nr, sem = bufs[b]
        sl = pl.ds(j * jnp.int32(_CHUNK), _CHUNK)
        return (pltpu.make_async_copy(z_hbm.at[srci_v.at[sl]], sr, sem),
                pltpu.make_async_copy(z_hbm.at[dsti_v.at[sl]], dr, sem),
                pltpu.make_async_copy(z_hbm.at[negi_v.at[sl]], nr, sem))

    def issue(j, b):
        for c in gather_descs(j, b):
            c.start()

    def drain(j, b):
        for c in gather_descs(j, b):
            c.wait()

    def compute(j, b):
        sr, dr, nr, _ = bufs[b]
        ebase = j * jnp.int32(_CHUNK)

        @plsc.parallel_loop(jnp.int32(0), jnp.int32(_CHUNK),
                            jnp.int32(_UNROLL), unroll=2)
        def edge_body(e0):
            for i in range(_UNROLL):
                e = e0 + jnp.int32(i)
                accp = None
                accn = None
                for k in range(_D // _L):
                    ks = pl.ds(k * _L, _L)
                    s = sr[e, ks]
                    pp = s * dr[e, ks]
                    pn = s * nr[e, ks]
                    accp = pp if accp is None else accp + pp
                    accn = pn if accn is None else accn + pn
                p1 = accp + accp[perms[1]]
                n1 = accn + accn[perms[1]]
                c = jnp.where(even, p1, n1)
                for sh in (2, 4, 8):
                    c = c + c[perms[sh]]
                ev = jnp.full((_L,), ebase + e, jnp.int32)
                plsc.store_scatter(posacc, [ev], c, mask=lane0)
                plsc.store_scatter(negacc, [ev], c, mask=lane1)

    for b in range(_NBUF):
        issue(jnp.int32(b), b)

    def round_body(t, c):
        j0 = t * jnp.int32(_NBUF)
        for b in range(_NBUF):
            j = j0 + jnp.int32(b)
            drain(j, b)
            compute(j, b)

            @pl.when(j + _NBUF < _NCHUNK)
            def _():
                issue(j + jnp.int32(_NBUF), b)

        return c

    lax.fori_loop(jnp.int32(0), jnp.int32(_NCHUNK // _NBUF), round_body,
                  jnp.int32(0))
    for b in range(_NCHUNK % _NBUF):
        jt = jnp.int32((_NCHUNK // _NBUF) * _NBUF + b)
        drain(jt, b)
        compute(jt, b)

    pltpu.sync_copy(posacc, pos_hbm.at[pl.ds(base, _EW)])
    pltpu.sync_copy(negacc, negv_hbm.at[pl.ds(base, _EW)])


def _tc_reduce_body(pos_ref, neg_ref, out_ref):
    p = pos_ref[...]
    q = neg_ref[...]
    pos_term = -jnp.log(jax.nn.sigmoid(p) + _EPS)
    neg_term = -jnp.log(1.0 - jax.nn.sigmoid(q) + _EPS)
    out_ref[0] = (jnp.sum(pos_term) + jnp.sum(neg_term)) / _E


_tc_reduce = pl.pallas_call(
    _tc_reduce_body,
    out_shape=jax.ShapeDtypeStruct((1,), jnp.float32),
    out_specs=pl.BlockSpec(memory_space=pltpu.SMEM),
)


def _threefry2x32_np(k1, k2, x0, x1):
    rotations = [(13, 15, 26, 6), (17, 29, 16, 24)]
    ks = [np.uint32(k1), np.uint32(k2),
          np.uint32(np.uint32(k1) ^ np.uint32(k2) ^ np.uint32(0x1BD11BDA))]
    x0 = x0 + ks[0]
    x1 = x1 + ks[1]

    def rotl(x, d):
        return (x << np.uint32(d)) | (x >> np.uint32(32 - d))

    for i in range(5):
        for r in rotations[i % 2]:
            x0 = x0 + x1
            x1 = x0 ^ rotl(x1, r)
        x0 = x0 + ks[(i + 1) % 3]
        x1 = x1 + ks[(i + 2) % 3] + np.uint32(i + 1)
    return x0, x1


def _make_u() -> np.ndarray:
    i = np.arange(_E, dtype=np.uint64)
    x0, x1 = _threefry2x32_np(0, 42,
                              (i >> np.uint64(32)).astype(np.uint32),
                              (i & np.uint64(0xFFFFFFFF)).astype(np.uint32))
    bits = x0 ^ x1
    fbits = (bits >> np.uint32(9)) | np.uint32(0x3F800000)
    return fbits.view(np.float32) - np.float32(1.0)


_U = _make_u()


def kernel(z, edge_index, batch):
    z32 = z.astype(jnp.float32)
    ei32 = jax.lax.bitcast_convert_type(edge_index, jnp.int32)
    b32 = batch.astype(jnp.int32)
    counts = jnp.sum(
        b32[None, :] == jnp.arange(_NUM_GRAPHS, dtype=jnp.int32)[:, None],
        axis=1, dtype=jnp.int32)
    starts = (jnp.cumsum(counts) - counts).astype(jnp.int32)
    cnt_tab = jnp.maximum(counts, 1).astype(jnp.int32)
    u = jnp.asarray(_U)
    pos_v, neg_v = _sc_dots(z32, ei32[0].reshape(2 * _E), ei32[1].reshape(2 * _E),
                            u, cnt_tab, starts)
    loss = _tc_reduce(pos_v.reshape(_E // _D, _D), neg_v.reshape(_E // _D, _D))
    return loss[0]


# speedup vs baseline: 2.5320x; 2.5320x over previous
import functools

import jax
import jax.numpy as jnp
import numpy as np
from jax import lax
from jax.experimental import pallas as pl
from jax.experimental.pallas import tpu as pltpu
from jax.experimental.pallas import tpu_sc as plsc

_EPS = 1e-05
_NUM_GRAPHS = 16
_N, _D, _E = 10000, 128, 320000
_NC, _NS, _L = 2, 16, 16
_NW = _NC * _NS
_EW = _E // _NW
_CHUNK = 40
_NCHUNK = _EW // _CHUNK
_NBUF = 4
_UNROLL = 2

_ROW_SCRATCH = [pltpu.VMEM((_CHUNK, _D), jnp.float32)] * (3 * _NBUF)


def _sc_mesh():
    return plsc.VectorSubcoreMesh(core_axis_name="c", subcore_axis_name="s",
                                  num_cores=_NC, num_subcores=_NS)


@functools.partial(
    pl.kernel,
    out_type=[jax.ShapeDtypeStruct((_E,), jnp.float32),
              jax.ShapeDtypeStruct((_E,), jnp.float32)],
    mesh=_sc_mesh(),
    scratch_types=[
        pltpu.VMEM((_NUM_GRAPHS,), jnp.int32),
        pltpu.VMEM((_NUM_GRAPHS,), jnp.int32),
        pltpu.VMEM((_EW,), jnp.int32),
        pltpu.VMEM((_EW,), jnp.int32),
        pltpu.VMEM((_EW,), jnp.int32),
        pltpu.VMEM((_EW,), jnp.float32),
        pltpu.VMEM((_EW,), jnp.float32),
        pltpu.VMEM((_EW,), jnp.float32),
    ] + _ROW_SCRATCH + [
        pltpu.SemaphoreType.DMA,
    ] + [pltpu.SemaphoreType.DMA] * _NBUF,
    compiler_params=pltpu.CompilerParams(needs_layout_passes=False),
)
def _sc_dots(z_hbm, src_hbm, dst_hbm, u_hbm, cnt_hbm, start_hbm,
             pos_hbm, negv_hbm,
             cnt_v, start_v, srci_v, dsti_v, negi_v, u_v, posacc, negacc,
             *rest):
    rows = rest[:3 * _NBUF]
    sem_in = rest[3 * _NBUF]
    sems = rest[3 * _NBUF + 1:]
    bufs = tuple((rows[3 * b], rows[3 * b + 1], rows[3 * b + 2], sems[b])
                 for b in range(_NBUF))

    wid = lax.axis_index("s") * jnp.int32(_NC) + lax.axis_index("c")
    base = wid * jnp.int32(_EW)
    lane = lax.iota(jnp.int32, _L)
    lane0 = lane == 0
    lane1 = lane == 1
    even = (lane & jnp.int32(1)) == 0
    perms = {sh: lane ^ jnp.int32(sh) for sh in (1, 2, 4, 8)}

    pltpu.sync_copy(cnt_hbm, cnt_v)
    pltpu.sync_copy(start_hbm, start_v)
    cp_a = pltpu.async_copy(src_hbm.at[pl.ds(base, _EW)], srci_v, sem_in)
    cp_b = pltpu.async_copy(dst_hbm.at[pl.ds(base, _EW)], dsti_v, sem_in)
    cp_c = pltpu.async_copy(u_hbm.at[pl.ds(base, _EW)], u_v, sem_in)
    cp_a.wait()
    cp_b.wait()
    cp_c.wait()

    def neg_body(t, c):
        sl = pl.ds(t * jnp.int32(_L), _L)
        sv = srci_v[sl]
        g = jnp.zeros((_L,), jnp.int32)
        for bit in (8, 4, 2, 1):
            probe = g | jnp.int32(bit)
            vals = plsc.load_gather(start_v, [probe])
            g = jnp.where(vals <= sv, probe, g)
        cnt = plsc.load_gather(cnt_v, [g])
        st = plsc.load_gather(start_v, [g])
        r = (u_v[sl] * cnt.astype(jnp.float32)).astype(jnp.int32)
        negi_v[sl] = st + jnp.minimum(r, cnt - 1)
        return c

    lax.fori_loop(jnp.int32(0), jnp.int32(_EW // _L), neg_body, jnp.int32(0))

    def gather_descs(j, b):
        sr, dr, nr, sem = bufs[b]
        sl = pl.ds(j * jnp.int32(_CHUNK), _CHUNK)
        return (pltpu.make_async_copy(z_hbm.at[srci_v.at[sl]], sr, sem),
                pltpu.make_async_copy(z_hbm.at[dsti_v.at[sl]], dr, sem),
                pltpu.make_async_copy(z_hbm.at[negi_v.at[sl]], nr, sem))

    def issue(j, b):
        for c in gather_descs(j, b):
            c.start()

    def drain(j, b):
        for c in gather_descs(j, b):
            c.wait()

    def compute(j, b):
        sr, dr, nr, _ = bufs[b]
        ebase = j * jnp.int32(_CHUNK)

        @plsc.parallel_loop(jnp.int32(0), jnp.int32(_CHUNK),
                            jnp.int32(_UNROLL), unroll=2)
        def edge_body(e0):
            for i in range(_UNROLL):
                e = e0 + jnp.int32(i)
                accp = None
                accn = None
                for k in range(_D // _L):
                    ks = pl.ds(k * _L, _L)
                    s = sr[e, ks]
                    pp = s * dr[e, ks]
                    pn = s * nr[e, ks]
                    accp = pp if accp is None else accp + pp
                    accn = pn if accn is None else accn + pn
                p1 = accp + accp[perms[1]]
                n1 = accn + accn[perms[1]]
                c = jnp.where(even, p1, n1)
                for sh in (2, 4, 8):
                    c = c + c[perms[sh]]
                ev = jnp.full((_L,), ebase + e, jnp.int32)
                plsc.store_scatter(posacc, [ev], c, mask=lane0)
                plsc.store_scatter(negacc, [ev], c, mask=lane1)

    for b in range(_NBUF):
        issue(jnp.int32(b), b)

    def round_body(t, c):
        j0 = t * jnp.int32(_NBUF)
        for b in range(_NBUF):
            j = j0 + jnp.int32(b)
            drain(j, b)
            compute(j, b)

            @pl.when(j + _NBUF < _NCHUNK)
            def _():
                issue(j + jnp.int32(_NBUF), b)

        return c

    lax.fori_loop(jnp.int32(0), jnp.int32(_NCHUNK // _NBUF), round_body,
                  jnp.int32(0))
    for b in range(_NCHUNK % _NBUF):
        jt = jnp.int32((_NCHUNK // _NBUF) * _NBUF + b)
        drain(jt, b)
        compute(jt, b)

    pltpu.sync_copy(posacc, pos_hbm.at[pl.ds(base, _EW)])
    pltpu.sync_copy(negacc, negv_hbm.at[pl.ds(base, _EW)])


def _tc_reduce_body(pos_ref, neg_ref, out_ref):
    p = pos_ref[...]
    q = neg_ref[...]
    pos_term = -jnp.log(jax.nn.sigmoid(p) + _EPS)
    neg_term = -jnp.log(1.0 - jax.nn.sigmoid(q) + _EPS)
    out_ref[0] = (jnp.sum(pos_term) + jnp.sum(neg_term)) / _E


_tc_reduce = pl.pallas_call(
    _tc_reduce_body,
    out_shape=jax.ShapeDtypeStruct((1,), jnp.float32),
    out_specs=pl.BlockSpec(memory_space=pltpu.SMEM),
)


def _threefry2x32_np(k1, k2, x0, x1):
    rotations = [(13, 15, 26, 6), (17, 29, 16, 24)]
    ks = [np.uint32(k1), np.uint32(k2),
          np.uint32(np.uint32(k1) ^ np.uint32(k2) ^ np.uint32(0x1BD11BDA))]
    x0 = x0 + ks[0]
    x1 = x1 + ks[1]

    def rotl(x, d):
        return (x << np.uint32(d)) | (x >> np.uint32(32 - d))

    for i in range(5):
        for r in rotations[i % 2]:
            x0 = x0 + x1
            x1 = x0 ^ rotl(x1, r)
        x0 = x0 + ks[(i + 1) % 3]
        x1 = x1 + ks[(i + 2) % 3] + np.uint32(i + 1)
    return x0, x1


def _make_u() -> np.ndarray:
    i = np.arange(_E, dtype=np.uint64)
    x0, x1 = _threefry2x32_np(0, 42,
                              (i >> np.uint64(32)).astype(np.uint32),
                              (i & np.uint64(0xFFFFFFFF)).astype(np.uint32))
    bits = x0 ^ x1
    fbits = (bits >> np.uint32(9)) | np.uint32(0x3F800000)
    return fbits.view(np.float32) - np.float32(1.0)


_U = _make_u()


def kernel(z, edge_index, batch):
    z32 = z.astype(jnp.float32)
    src = edge_index[0].astype(jnp.int32)
    dst = edge_index[1].astype(jnp.int32)
    b32 = batch.astype(jnp.int32)
    counts = jnp.sum(
        b32[None, :] == jnp.arange(_NUM_GRAPHS, dtype=jnp.int32)[:, None],
        axis=1, dtype=jnp.int32)
    starts = (jnp.cumsum(counts) - counts).astype(jnp.int32)
    cnt_tab = jnp.maximum(counts, 1).astype(jnp.int32)
    u = jnp.asarray(_U)
    pos_v, neg_v = _sc_dots(z32, src, dst, u, cnt_tab, starts)
    loss = _tc_reduce(pos_v.reshape(_E // _D, _D), neg_v.reshape(_E // _D, _D))
    return loss[0]


# speedup vs baseline: 2.5509x; 1.0075x over previous
import functools

import jax
import jax.numpy as jnp
import numpy as np
from jax import lax
from jax.experimental import pallas as pl
from jax.experimental.pallas import tpu as pltpu
from jax.experimental.pallas import tpu_sc as plsc

_EPS = 1e-05
_NUM_GRAPHS = 16
_N, _D, _E = 10000, 128, 320000
_NC, _NS, _L = 2, 16, 16
_NW = _NC * _NS
_EW = _E // _NW
_CHUNK = 40
_NCHUNK = _EW // _CHUNK
_NBUF = 4
_UNROLL = 2

_ROW_SCRATCH = [pltpu.VMEM((_CHUNK, _D), jnp.float32)] * (3 * _NBUF)


def _sc_mesh():
    return plsc.VectorSubcoreMesh(core_axis_name="c", subcore_axis_name="s",
                                  num_cores=_NC, num_subcores=_NS)


@functools.partial(
    pl.kernel,
    out_type=[jax.ShapeDtypeStruct((_E,), jnp.float32),
              jax.ShapeDtypeStruct((_E,), jnp.float32)],
    mesh=_sc_mesh(),
    scratch_types=[
        pltpu.VMEM((_NUM_GRAPHS,), jnp.int32),
        pltpu.VMEM((_NUM_GRAPHS,), jnp.int32),
        pltpu.VMEM((_EW,), jnp.int32),
        pltpu.VMEM((_EW,), jnp.int32),
        pltpu.VMEM((_EW,), jnp.int32),
        pltpu.VMEM((_EW,), jnp.float32),
        pltpu.VMEM((_EW,), jnp.float32),
        pltpu.VMEM((_EW,), jnp.float32),
    ] + _ROW_SCRATCH + [
        pltpu.SemaphoreType.DMA,
    ] + [pltpu.SemaphoreType.DMA] * _NBUF,
    compiler_params=pltpu.CompilerParams(needs_layout_passes=False),
)
def _sc_dots(z_hbm, src_hbm, dst_hbm, u_hbm, cnt_hbm, start_hbm,
             pos_hbm, negv_hbm,
             cnt_v, start_v, srci_v, dsti_v, negi_v, u_v, posacc, negacc,
             *rest):
    rows = rest[:3 * _NBUF]
    sem_in = rest[3 * _NBUF]
    sems = rest[3 * _NBUF + 1:]
    bufs = tuple((rows[3 * b], rows[3 * b + 1], rows[3 * b + 2], sems[b])
                 for b in range(_NBUF))

    wid = lax.axis_index("s") * jnp.int32(_NC) + lax.axis_index("c")
    base = wid * jnp.int32(_EW)
    lane = lax.iota(jnp.int32, _L)
    lane0 = lane == 0
    lane1 = lane == 1
    even = (lane & jnp.int32(1)) == 0
    perms = {sh: lane ^ jnp.int32(sh) for sh in (1, 2, 4, 8)}

    pltpu.sync_copy(cnt_hbm, cnt_v)
    pltpu.sync_copy(start_hbm, start_v)
    cp_a = pltpu.async_copy(src_hbm.at[pl.ds(base, _EW)], srci_v, sem_in)
    cp_b = pltpu.async_copy(dst_hbm.at[pl.ds(base, _EW)], dsti_v, sem_in)
    cp_c = pltpu.async_copy(u_hbm.at[pl.ds(base, _EW)], u_v, sem_in)
    cp_a.wait()
    cp_b.wait()
    cp_c.wait()

    def neg_body(t, c):
        sl = pl.ds(t * jnp.int32(_L), _L)
        sv = srci_v[sl]
        g = jnp.zeros((_L,), jnp.int32)
        for bit in (8, 4, 2, 1):
            probe = g | jnp.int32(bit)
            vals = plsc.load_gather(start_v, [probe])
            g = jnp.where(vals <= sv, probe, g)
        cnt = plsc.load_gather(cnt_v, [g])
        st = plsc.load_gather(start_v, [g])
        r = (u_v[sl] * cnt.astype(jnp.float32)).astype(jnp.int32)
        negi_v[sl] = st + jnp.minimum(r, cnt - 1)
        return c

    def gather_descs(j, b):
        sr, dr, nr, sem = bufs[b]
        sl = pl.ds(j * jnp.int32(_CHUNK), _CHUNK)
        return (pltpu.make_async_copy(z_hbm.at[srci_v.at[sl]], sr, sem),
                pltpu.make_async_copy(z_hbm.at[dsti_v.at[sl]], dr, sem),
                pltpu.make_async_copy(z_hbm.at[negi_v.at[sl]], nr, sem))

    def issue(j, b):
        for c in gather_descs(j, b):
            c.start()

    def drain(j, b):
        for c in gather_descs(j, b):
            c.wait()

    def compute(j, b):
        sr, dr, nr, _ = bufs[b]
        ebase = j * jnp.int32(_CHUNK)

        @plsc.parallel_loop(jnp.int32(0), jnp.int32(_CHUNK),
                            jnp.int32(_UNROLL), unroll=2)
        def edge_body(e0):
            for i in range(_UNROLL):
                e = e0 + jnp.int32(i)
                accp = None
                accn = None
                for k in range(_D // _L):
                    ks = pl.ds(k * _L, _L)
                    s = sr[e, ks]
                    pp = s * dr[e, ks]
                    pn = s * nr[e, ks]
                    accp = pp if accp is None else accp + pp
                    accn = pn if accn is None else accn + pn
                p1 = accp + accp[perms[1]]
                n1 = accn + accn[perms[1]]
                c = jnp.where(even, p1, n1)
                for sh in (2, 4, 8):
                    c = c + c[perms[sh]]
                ev = jnp.full((_L,), ebase + e, jnp.int32)
                plsc.store_scatter(posacc, [ev], c, mask=lane0)
                plsc.store_scatter(negacc, [ev], c, mask=lane1)

    for b in range(_NBUF):
        for c in gather_descs(jnp.int32(b), b)[:2]:
            c.start()
    lax.fori_loop(jnp.int32(0), jnp.int32(_EW // _L), neg_body, jnp.int32(0))
    for b in range(_NBUF):
        gather_descs(jnp.int32(b), b)[2].start()

    def round_body(t, c):
        j0 = t * jnp.int32(_NBUF)
        for b in range(_NBUF):
            j = j0 + jnp.int32(b)
            drain(j, b)
            compute(j, b)

            @pl.when(j + _NBUF < _NCHUNK)
            def _():
                issue(j + jnp.int32(_NBUF), b)

        return c

    lax.fori_loop(jnp.int32(0), jnp.int32(_NCHUNK // _NBUF), round_body,
                  jnp.int32(0))
    for b in range(_NCHUNK % _NBUF):
        jt = jnp.int32((_NCHUNK // _NBUF) * _NBUF + b)
        drain(jt, b)
        compute(jt, b)

    pltpu.sync_copy(posacc, pos_hbm.at[pl.ds(base, _EW)])
    pltpu.sync_copy(negacc, negv_hbm.at[pl.ds(base, _EW)])


def _tc_reduce_body(pos_ref, neg_ref, out_ref):
    p = pos_ref[...]
    q = neg_ref[...]
    pos_term = -jnp.log(jax.nn.sigmoid(p) + _EPS)
    neg_term = -jnp.log(1.0 - jax.nn.sigmoid(q) + _EPS)
    out_ref[0] = (jnp.sum(pos_term) + jnp.sum(neg_term)) / _E


_tc_reduce = pl.pallas_call(
    _tc_reduce_body,
    out_shape=jax.ShapeDtypeStruct((1,), jnp.float32),
    out_specs=pl.BlockSpec(memory_space=pltpu.SMEM),
)


def _threefry2x32_np(k1, k2, x0, x1):
    rotations = [(13, 15, 26, 6), (17, 29, 16, 24)]
    ks = [np.uint32(k1), np.uint32(k2),
          np.uint32(np.uint32(k1) ^ np.uint32(k2) ^ np.uint32(0x1BD11BDA))]
    x0 = x0 + ks[0]
    x1 = x1 + ks[1]

    def rotl(x, d):
        return (x << np.uint32(d)) | (x >> np.uint32(32 - d))

    for i in range(5):
        for r in rotations[i % 2]:
            x0 = x0 + x1
            x1 = x0 ^ rotl(x1, r)
        x0 = x0 + ks[(i + 1) % 3]
        x1 = x1 + ks[(i + 2) % 3] + np.uint32(i + 1)
    return x0, x1


def _make_u() -> np.ndarray:
    i = np.arange(_E, dtype=np.uint64)
    x0, x1 = _threefry2x32_np(0, 42,
                              (i >> np.uint64(32)).astype(np.uint32),
                              (i & np.uint64(0xFFFFFFFF)).astype(np.uint32))
    bits = x0 ^ x1
    fbits = (bits >> np.uint32(9)) | np.uint32(0x3F800000)
    return fbits.view(np.float32) - np.float32(1.0)


_U = _make_u()


def kernel(z, edge_index, batch):
    z32 = z.astype(jnp.float32)
    src = edge_index[0].astype(jnp.int32)
    dst = edge_index[1].astype(jnp.int32)
    b32 = batch.astype(jnp.int32)
    counts = jnp.sum(
        b32[None, :] == jnp.arange(_NUM_GRAPHS, dtype=jnp.int32)[:, None],
        axis=1, dtype=jnp.int32)
    starts = (jnp.cumsum(counts) - counts).astype(jnp.int32)
    cnt_tab = jnp.maximum(counts, 1).astype(jnp.int32)
    u = jnp.asarray(_U)
    pos_v, neg_v = _sc_dots(z32, src, dst, u, cnt_tab, starts)
    loss = _tc_reduce(pos_v.reshape(_E // _D, _D), neg_v.reshape(_E // _D, _D))
    return loss[0]
